# baseline (speedup 1.0000x reference)
import jax
import jax.numpy as jnp
from jax.experimental import pallas as pl
from jax.experimental.pallas import tpu as pltpu

_B, _SEQ, _C, _H = 32, 4096, 192, 4
_W = _SEQ // _H
_FIX = 384
_OVL = 4
_NVQ = 6
_K = 1024
_VD = 256
_COMMIT = 0.25
_NROW = _B * _W // _OVL
_JB = _W // _OVL
_BSUB = 2
_RT = 128


def _fused_kernel(z_ref, wd_ref, wu_ref, e_ref, out_ref, loss_ref,
                  eb_ref, wdt_ref, wut_ref, e2_ref):
    @pl.when(pl.program_id(0) == 0)
    def _prep():
        eb_ref[...] = e_ref[...].astype(jnp.bfloat16)
        for i in range(_NVQ):
            ei = e_ref[i]
            e2_ref[i, 0:1, :] = jnp.sum(ei * ei, axis=0, keepdims=True)
        rows = jax.lax.broadcasted_iota(jnp.int32, (_C * _H, _C), 0)
        cols = jax.lax.broadcasted_iota(jnp.int32, (_C * _H, _C), 1)
        for h in range(_H):
            sel = (rows == cols * _H + h).astype(jnp.float32)
            wdt_ref[h] = jax.lax.dot_general(
                sel, wd_ref[...], (((0,), (0,)), ((), ())),
                preferred_element_type=jnp.float32).astype(jnp.bfloat16)
            wut_ref[:, h * _C:(h + 1) * _C] = jnp.dot(
                wu_ref[...], sel,
                preferred_element_type=jnp.float32).astype(jnp.bfloat16)

    total = jnp.float32(0.0)
    for bb in range(_BSUB):
        zp = jnp.zeros((_W, _FIX), jnp.float32)
        for h in range(_H):
            zb = z_ref[bb, h * _W:(h + 1) * _W, :].astype(jnp.bfloat16)
            zp = zp + jnp.dot(zb, wdt_ref[h],
                              preferred_element_type=jnp.float32)
        total = total + jnp.sum(zp * zp)
        zo = (zp * -2.0).astype(jnp.bfloat16).reshape(_JB, _OVL * _FIX)

        zq_cols = []
        for i in range(_NVQ):
            zq_tiles = []
            for t in range(_JB // _RT):
                zfb = zo[t * _RT:(t + 1) * _RT, i * _VD:(i + 1) * _VD]
                q = (jax.lax.dot_general(zfb, eb_ref[i],
                                         (((1,), (0,)), ((), ())),
                                         preferred_element_type=jnp.float32)
                     + e2_ref[i, 0:1, :])
                m = jnp.min(q, axis=1, keepdims=True)
                total = total + jnp.sum(m)
                oh32 = (q == m).astype(jnp.float32)
                cnt = jnp.sum(oh32, axis=1, keepdims=True)
                onehot = oh32.astype(jnp.bfloat16)
                zq32 = jax.lax.dot_general(onehot, eb_ref[i],
                                           (((1,), (1,)), ((), ())),
                                           preferred_element_type=jnp.float32)
                zq_tiles.append((zq32 * (1.0 / cnt)).astype(jnp.bfloat16))
            zq_cols.append(jnp.concatenate(zq_tiles, axis=0))

        zq = jnp.concatenate(zq_cols, axis=1)
        zqp = zq.reshape(_W, _FIX)

        for h in range(_H):
            out_ref[bb, h * _W:(h + 1) * _W, :] = jnp.dot(
                zqp, wut_ref[:, h * _C:(h + 1) * _C],
                preferred_element_type=jnp.float32)

    @pl.when(pl.program_id(0) == 0)
    def _init():
        loss_ref[...] = jnp.full((8, 128), total, jnp.float32)

    @pl.when(pl.program_id(0) != 0)
    def _acc():
        loss_ref[...] = loss_ref[...] + jnp.full((8, 128), total, jnp.float32)


def kernel(z, Wd, Wu, E):
    out, lossb = pl.pallas_call(
        _fused_kernel,
        grid=(_B // _BSUB,),
        in_specs=[pl.BlockSpec((_BSUB, _SEQ, _C), lambda b: (b, 0, 0)),
                  pl.BlockSpec((_C * _H, _FIX), lambda b: (0, 0)),
                  pl.BlockSpec((_FIX, _C * _H), lambda b: (0, 0)),
                  pl.BlockSpec((_NVQ, _VD, _K), lambda b: (0, 0, 0))],
        out_specs=[pl.BlockSpec((_BSUB, _SEQ, _C), lambda b: (b, 0, 0)),
                   pl.BlockSpec((8, 128), lambda b: (0, 0))],
        out_shape=[jax.ShapeDtypeStruct((_B, _SEQ, _C), jnp.float32),
                   jax.ShapeDtypeStruct((8, 128), jnp.float32)],
        scratch_shapes=[pltpu.VMEM((_NVQ, _VD, _K), jnp.bfloat16),
                        pltpu.VMEM((_H, _C, _FIX), jnp.bfloat16),
                        pltpu.VMEM((_FIX, _C * _H), jnp.bfloat16),
                        pltpu.VMEM((_NVQ, 8, _K), jnp.float32)],
    )(z, Wd, Wu, E)

    loss = lossb[0, 0] * (_COMMIT / (_NROW * _VD * _NVQ))
    return out, loss


# speedup vs baseline: 1.0760x; 1.0760x over previous
import jax
import jax.numpy as jnp
from jax.experimental import pallas as pl
from jax.experimental.pallas import tpu as pltpu

_B, _SEQ, _C, _H = 32, 4096, 192, 4
_W = _SEQ // _H
_FIX = 384
_OVL = 4
_NVQ = 6
_K = 1024
_VD = 256
_COMMIT = 0.25
_NROW = _B * _W // _OVL
_JB = _W // _OVL
_BSUB = 2
_RT = 128


def _fused_kernel(z_ref, wd_ref, wu_ref, e_ref, out_ref, loss_ref,
                  eb_ref, wdt_ref, wut_ref, e2_ref):
    @pl.when(pl.program_id(0) == 0)
    def _prep():
        eb_ref[...] = e_ref[...].astype(jnp.bfloat16)
        for i in range(_NVQ):
            ei = e_ref[i]
            e2_ref[i, 0:1, :] = jnp.sum(ei * ei, axis=0, keepdims=True)
        rows = jax.lax.broadcasted_iota(jnp.int32, (_C * _H, _C), 0)
        cols = jax.lax.broadcasted_iota(jnp.int32, (_C * _H, _C), 1)
        for h in range(_H):
            sel = (rows == cols * _H + h).astype(jnp.float32)
            wdt_ref[h] = jax.lax.dot_general(
                sel, wd_ref[...], (((0,), (0,)), ((), ())),
                preferred_element_type=jnp.float32).astype(jnp.bfloat16)
            wut_ref[:, h * _C:(h + 1) * _C] = jnp.dot(
                wu_ref[...], sel,
                preferred_element_type=jnp.float32).astype(jnp.bfloat16)

    total = jnp.float32(0.0)
    for bb in range(_BSUB):
        zp = jnp.zeros((_W, _FIX), jnp.float32)
        for h in range(_H):
            zb = z_ref[bb, h * _W:(h + 1) * _W, :].astype(jnp.bfloat16)
            zp = zp + jnp.dot(zb, wdt_ref[h],
                              preferred_element_type=jnp.float32)
        total = total + jnp.sum(zp * zp)
        zo = (zp * -2.0).astype(jnp.bfloat16).reshape(_JB, _OVL * _FIX)

        zq_cols = []
        for i in range(_NVQ):
            zfb = zo[:, i * _VD:(i + 1) * _VD]
            q = (jax.lax.dot_general(zfb, eb_ref[i],
                                     (((1,), (0,)), ((), ())),
                                     preferred_element_type=jnp.float32)
                 + e2_ref[i, 0:1, :])
            m = jnp.min(q, axis=1, keepdims=True)
            total = total + jnp.sum(m)
            oh32 = (q == m).astype(jnp.float32)
            cnt = jnp.sum(oh32, axis=1, keepdims=True)
            onehot = oh32.astype(jnp.bfloat16)
            zq32 = jax.lax.dot_general(onehot, eb_ref[i],
                                       (((1,), (1,)), ((), ())),
                                       preferred_element_type=jnp.float32)
            zq_cols.append((zq32 * (1.0 / cnt)).astype(jnp.bfloat16))

        zq = jnp.concatenate(zq_cols, axis=1)
        zqp = zq.reshape(_W, _FIX)

        for h in range(_H):
            out_ref[bb, h * _W:(h + 1) * _W, :] = jnp.dot(
                zqp, wut_ref[:, h * _C:(h + 1) * _C],
                preferred_element_type=jnp.float32)

    @pl.when(pl.program_id(0) == 0)
    def _init():
        loss_ref[...] = jnp.full((8, 128), total, jnp.float32)

    @pl.when(pl.program_id(0) != 0)
    def _acc():
        loss_ref[...] = loss_ref[...] + jnp.full((8, 128), total, jnp.float32)


def kernel(z, Wd, Wu, E):
    out, lossb = pl.pallas_call(
        _fused_kernel,
        grid=(_B // _BSUB,),
        in_specs=[pl.BlockSpec((_BSUB, _SEQ, _C), lambda b: (b, 0, 0)),
                  pl.BlockSpec((_C * _H, _FIX), lambda b: (0, 0)),
                  pl.BlockSpec((_FIX, _C * _H), lambda b: (0, 0)),
                  pl.BlockSpec((_NVQ, _VD, _K), lambda b: (0, 0, 0))],
        out_specs=[pl.BlockSpec((_BSUB, _SEQ, _C), lambda b: (b, 0, 0)),
                   pl.BlockSpec((8, 128), lambda b: (0, 0))],
        out_shape=[jax.ShapeDtypeStruct((_B, _SEQ, _C), jnp.float32),
                   jax.ShapeDtypeStruct((8, 128), jnp.float32)],
        scratch_shapes=[pltpu.VMEM((_NVQ, _VD, _K), jnp.bfloat16),
                        pltpu.VMEM((_H, _C, _FIX), jnp.bfloat16),
                        pltpu.VMEM((_FIX, _C * _H), jnp.bfloat16),
                        pltpu.VMEM((_NVQ, 8, _K), jnp.float32)],
    )(z, Wd, Wu, E)

    loss = lossb[0, 0] * (_COMMIT / (_NROW * _VD * _NVQ))
    return out, loss
